# baseline (speedup 1.0000x reference)
import functools

import jax
import jax.numpy as jnp
from jax import lax
from jax.experimental import pallas as pl
from jax.experimental.pallas import tpu as pltpu
from jax.experimental.pallas import tpu_sc as plsc

N = 10000
D = 128
NUM_CLASSES = 64
NC = 2
NS = 16
NW = NC * NS
CHUNK = 128
PHASE = 40
NPAD = 10240
STRIPE = NPAD // NS

_MESH = plsc.VectorSubcoreMesh(core_axis_name="c", subcore_axis_name="s")


def _sc_degree(dst_p, n_chunks):
    @functools.partial(
        pl.kernel,
        out_type=jax.ShapeDtypeStruct((NC, NPAD), jnp.float32),
        mesh=_MESH,
        scratch_types=[
            pltpu.VMEM((n_chunks, CHUNK), jnp.int32),
            pltpu.VMEM((CHUNK,), jnp.float32),
            pltpu.VMEM((STRIPE,), jnp.float32),
            pltpu.VMEM_SHARED((NPAD,), jnp.float32),
        ],
    )
    def deg_kernel(dst_hbm, out_hbm, dst_v, ones_v, zb_v, acc_sh):
        cid = lax.axis_index("c")
        sid = lax.axis_index("s")
        wid = sid * NC + cid
        for k in range(CHUNK // 16):
            ones_v[pl.ds(k * 16, 16)] = jnp.ones((16,), jnp.float32)

        def zfill(i, carry):
            zb_v[pl.ds(i * 16, 16)] = jnp.zeros((16,), jnp.float32)
            return carry

        lax.fori_loop(0, STRIPE // 16, zfill, 0)
        pltpu.sync_copy(dst_hbm.at[wid], dst_v)
        pltpu.sync_copy(zb_v, acc_sh.at[pl.ds(sid * STRIPE, STRIPE)])
        plsc.subcore_barrier()

        def body(j, carry):
            pltpu.sync_copy(ones_v, acc_sh.at[dst_v.at[j]], add=True)
            return carry

        lax.fori_loop(0, n_chunks, body, 0)
        plsc.subcore_barrier()
        pltpu.sync_copy(acc_sh.at[pl.ds(sid * STRIPE, STRIPE)],
                        out_hbm.at[cid, pl.ds(sid * STRIPE, STRIPE)])

    return deg_kernel(dst_p)


def _sc_scatter(y, src_p, dst_p, n_chunks):
    assert n_chunks % PHASE == 0
    n_phases = n_chunks // PHASE

    @functools.partial(
        pl.kernel,
        out_type=jax.ShapeDtypeStruct((NC, NPAD, D), jnp.float32),
        mesh=_MESH,
        scratch_types=[
            pltpu.VMEM((PHASE, CHUNK), jnp.int32),
            pltpu.VMEM((PHASE, CHUNK), jnp.int32),
            pltpu.VMEM((CHUNK, D), jnp.float32),
            pltpu.VMEM((CHUNK, D), jnp.float32),
            pltpu.VMEM_SHARED((NPAD, D), jnp.float32),
            pltpu.SemaphoreType.DMA,
            pltpu.SemaphoreType.DMA,
        ],
    )
    def scatter_kernel(y_hbm, src_hbm, dst_hbm, out_hbm,
                       src_v, dst_v, rows0_v, rows1_v, acc_sh, sem0, sem1):
        cid = lax.axis_index("c")
        sid = lax.axis_index("s")
        wid = sid * NC + cid

        def zfill(i, carry):
            for k in range(D // 16):
                rows0_v[i, pl.ds(k * 16, 16)] = jnp.zeros((16,), jnp.float32)
            return carry

        lax.fori_loop(0, CHUNK, zfill, 0)
        for r in range(STRIPE // CHUNK):
            pltpu.sync_copy(
                rows0_v, acc_sh.at[pl.ds(sid * STRIPE + r * CHUNK, CHUNK)])
        plsc.subcore_barrier()

        for p in range(n_phases):
            pltpu.sync_copy(src_hbm.at[wid, pl.ds(p * PHASE, PHASE)], src_v)
            pltpu.sync_copy(dst_hbm.at[wid, pl.ds(p * PHASE, PHASE)], dst_v)
            pltpu.async_copy(y_hbm.at[src_v.at[0]], rows0_v, sem0)

            def body(i, carry):
                j = 2 * i
                pltpu.async_copy(y_hbm.at[src_v.at[j + 1]], rows1_v, sem1)
                pltpu.make_async_copy(
                    y_hbm.at[src_v.at[j]], rows0_v, sem0).wait()
                pltpu.sync_copy(rows0_v, acc_sh.at[dst_v.at[j]], add=True)
                pltpu.async_copy(y_hbm.at[src_v.at[j + 2]], rows0_v, sem0)
                pltpu.make_async_copy(
                    y_hbm.at[src_v.at[j + 1]], rows1_v, sem1).wait()
                pltpu.sync_copy(rows1_v, acc_sh.at[dst_v.at[j + 1]], add=True)
                return carry

            lax.fori_loop(0, (PHASE - 2) // 2, body, 0)
            pltpu.async_copy(y_hbm.at[src_v.at[PHASE - 1]], rows1_v, sem1)
            pltpu.make_async_copy(
                y_hbm.at[src_v.at[PHASE - 2]], rows0_v, sem0).wait()
            pltpu.sync_copy(rows0_v, acc_sh.at[dst_v.at[PHASE - 2]], add=True)
            pltpu.make_async_copy(
                y_hbm.at[src_v.at[PHASE - 1]], rows1_v, sem1).wait()
            pltpu.sync_copy(rows1_v, acc_sh.at[dst_v.at[PHASE - 1]], add=True)
        plsc.subcore_barrier()
        pltpu.sync_copy(acc_sh.at[pl.ds(sid * STRIPE, STRIPE)],
                        out_hbm.at[cid, pl.ds(sid * STRIPE, STRIPE)])

    return scatter_kernel(y, src_p, dst_p)


def _tc_first(x, W1, deg0, deg1):
    def body(x_ref, w_ref, d0_ref, d1_ref, y_ref, dis_ref):
        dis = lax.rsqrt(d0_ref[...] + d1_ref[...] + 1.0)
        dis_ref[...] = dis
        xw = jnp.dot(x_ref[...], w_ref[...], preferred_element_type=jnp.float32)
        y_ref[...] = xw * dis[:, None]

    return pl.pallas_call(
        body,
        out_shape=(jax.ShapeDtypeStruct((N, D), jnp.float32),
                   jax.ShapeDtypeStruct((N,), jnp.float32)),
    )(x, W1, deg0, deg1)


def _tc_mid(acc0, acc1, yprev, dis, b, W):
    def body(a0_ref, a1_ref, yp_ref, dis_ref, b_ref, w_ref, o_ref):
        dis = dis_ref[...][:, None]
        h = dis * (a0_ref[...] + a1_ref[...] + yp_ref[...]) + b_ref[...][None, :]
        h = jnp.maximum(h, 0.0)
        hw = jnp.dot(h, w_ref[...], preferred_element_type=jnp.float32)
        o_ref[...] = hw * dis

    return pl.pallas_call(
        body,
        out_shape=jax.ShapeDtypeStruct((N, D), jnp.float32),
    )(acc0, acc1, yprev, dis, b, W)


def _log_softmax(z):
    m = jnp.max(z, axis=1, keepdims=True)
    return z - (jnp.log(jnp.sum(jnp.exp(z - m), axis=1, keepdims=True)) + m)


def _tc_final(acc0, acc1, y3, dis, b3, prot, LW1, Lb1, LW2, Lb2):
    def body(a0_ref, a1_ref, y3_ref, dis_ref, b3_ref, prot_ref,
             lw1_ref, lb1_ref, lw2_ref, lb2_ref,
             out_ref, xrel_ref, oproto_ref):
        dis = dis_ref[...][:, None]
        h3 = dis * (a0_ref[...] + a1_ref[...] + y3_ref[...]) + b3_ref[...][None, :]
        qn = jnp.sqrt(jnp.sum(h3 * h3, axis=1, keepdims=True))
        hn = h3 / qn
        ids = lax.broadcasted_iota(jnp.int32, (NUM_CLASSES, N), 1)
        onehot = (ids == prot_ref[...][:, None]).astype(jnp.float32)
        anchors = jnp.dot(onehot, hn, preferred_element_type=jnp.float32)
        an = jnp.maximum(
            jnp.sqrt(jnp.sum(anchors * anchors, axis=1, keepdims=True)), 1e-6)
        xn = jnp.maximum(
            jnp.sqrt(jnp.sum(hn * hn, axis=1, keepdims=True)), 1e-6)
        xr = lax.dot_general(hn, anchors, (((1,), (1,)), ((), ())),
                             preferred_element_type=jnp.float32)
        xr = xr / (xn * an[:, 0][None, :])
        xrel_ref[...] = xr
        z = jnp.dot(anchors, lw1_ref[...], preferred_element_type=jnp.float32)
        z = jnp.maximum(z + lb1_ref[...][None, :], 0.0)
        z = jnp.dot(z, lw2_ref[...], preferred_element_type=jnp.float32)
        z = z + lb2_ref[...][None, :]
        op = _log_softmax(z)
        oproto_ref[...] = op
        o = jnp.dot(xr, op, preferred_element_type=jnp.float32)
        out_ref[...] = _log_softmax(o)

    return pl.pallas_call(
        body,
        out_shape=(jax.ShapeDtypeStruct((N, NUM_CLASSES), jnp.float32),
                   jax.ShapeDtypeStruct((N, NUM_CLASSES), jnp.float32),
                   jax.ShapeDtypeStruct((NUM_CLASSES, NUM_CLASSES), jnp.float32)),
    )(acc0, acc1, y3, dis, b3, prot, LW1, Lb1, LW2, Lb2)


def kernel(x, edge_index, epoch, prot, W1, b1, W2, b2, W3, b3,
           LW1, Lb1, LW2, Lb2):
    E = edge_index.shape[1]
    n_chunks = -(-E // (NW * CHUNK))
    n_chunks = -(-n_chunks // PHASE) * PHASE
    epad = NW * n_chunks * CHUNK
    src = edge_index[0]
    dst = edge_index[1]
    pad = epad - E
    pad_i = jnp.arange(pad, dtype=src.dtype)
    src_p = jnp.concatenate(
        [src, pad_i % N]).reshape(NW, n_chunks, CHUNK)
    dst_p = jnp.concatenate(
        [dst, N + pad_i % (NPAD - N)]).reshape(NW, n_chunks, CHUNK)

    degs = _sc_degree(dst_p, n_chunks)
    deg0 = degs[0, :N]
    deg1 = degs[1, :N]

    y1, dis = _tc_first(x, W1, deg0, deg1)
    acc = _sc_scatter(y1, src_p, dst_p, n_chunks)
    y2 = _tc_mid(acc[0, :N], acc[1, :N], y1, dis, b1, W2)
    acc = _sc_scatter(y2, src_p, dst_p, n_chunks)
    y3 = _tc_mid(acc[0, :N], acc[1, :N], y2, dis, b2, W3)
    acc = _sc_scatter(y3, src_p, dst_p, n_chunks)
    return _tc_final(acc[0, :N], acc[1, :N], y3, dis, b3, prot,
                     LW1, Lb1, LW2, Lb2)


# speedup vs baseline: 1.0207x; 1.0207x over previous
import functools

import jax
import jax.numpy as jnp
from jax import lax
from jax.experimental import pallas as pl
from jax.experimental.pallas import tpu as pltpu
from jax.experimental.pallas import tpu_sc as plsc

N = 10000
D = 128
NUM_CLASSES = 64
NC = 2
NS = 16
NW = NC * NS
CHUNK = 128
PHASE = 40
NPAD = 10240
STRIPE = NPAD // NS

_MESH = plsc.VectorSubcoreMesh(core_axis_name="c", subcore_axis_name="s")


def _sc_degree(dst_p, n_chunks):
    @functools.partial(
        pl.kernel,
        out_type=jax.ShapeDtypeStruct((NC, NPAD), jnp.float32),
        mesh=_MESH,
        scratch_types=[
            pltpu.VMEM((n_chunks, CHUNK), jnp.int32),
            pltpu.VMEM((CHUNK,), jnp.float32),
            pltpu.VMEM((STRIPE,), jnp.float32),
            pltpu.VMEM_SHARED((NPAD,), jnp.float32),
        ],
    )
    def deg_kernel(dst_hbm, out_hbm, dst_v, ones_v, zb_v, acc_sh):
        cid = lax.axis_index("c")
        sid = lax.axis_index("s")
        wid = sid * NC + cid
        for k in range(CHUNK // 16):
            ones_v[pl.ds(k * 16, 16)] = jnp.ones((16,), jnp.float32)

        def zfill(i, carry):
            zb_v[pl.ds(i * 16, 16)] = jnp.zeros((16,), jnp.float32)
            return carry

        lax.fori_loop(0, STRIPE // 16, zfill, 0)
        pltpu.sync_copy(dst_hbm.at[wid], dst_v)
        pltpu.sync_copy(zb_v, acc_sh.at[pl.ds(sid * STRIPE, STRIPE)])
        plsc.subcore_barrier()

        def body(j, carry):
            pltpu.sync_copy(ones_v, acc_sh.at[dst_v.at[j]], add=True)
            return carry

        lax.fori_loop(0, n_chunks, body, 0)
        plsc.subcore_barrier()
        pltpu.sync_copy(acc_sh.at[pl.ds(sid * STRIPE, STRIPE)],
                        out_hbm.at[cid, pl.ds(sid * STRIPE, STRIPE)])

    return deg_kernel(dst_p)


def _sc_scatter(y, src_p, dst_p, n_chunks):
    assert n_chunks % PHASE == 0
    n_phases = n_chunks // PHASE

    @functools.partial(
        pl.kernel,
        out_type=jax.ShapeDtypeStruct((NC, NPAD, D), jnp.float32),
        mesh=_MESH,
        scratch_types=[
            pltpu.VMEM((PHASE, CHUNK), jnp.int32),
            pltpu.VMEM((PHASE, CHUNK), jnp.int32),
            pltpu.VMEM((CHUNK, D), jnp.float32),
            pltpu.VMEM((CHUNK, D), jnp.float32),
            pltpu.VMEM_SHARED((NPAD, D), jnp.float32),
            pltpu.SemaphoreType.DMA,
            pltpu.SemaphoreType.DMA,
            pltpu.SemaphoreType.DMA,
        ],
    )
    def scatter_kernel(y_hbm, src_hbm, dst_hbm, out_hbm,
                       src_v, dst_v, rows0_v, rows1_v, acc_sh,
                       sem0, sem1, semz):
        cid = lax.axis_index("c")
        sid = lax.axis_index("s")
        wid = sid * NC + cid

        def zfill(i, carry):
            for k in range(D // 16):
                rows1_v[i, pl.ds(k * 16, 16)] = jnp.zeros((16,), jnp.float32)
            return carry

        lax.fori_loop(0, CHUNK, zfill, 0)
        for r in range(STRIPE // CHUNK):
            pltpu.async_copy(
                rows1_v, acc_sh.at[pl.ds(sid * STRIPE + r * CHUNK, CHUNK)],
                semz)
        pltpu.sync_copy(src_hbm.at[wid, pl.ds(0, PHASE)], src_v)
        pltpu.sync_copy(dst_hbm.at[wid, pl.ds(0, PHASE)], dst_v)
        pltpu.async_copy(y_hbm.at[src_v.at[0]], rows0_v, sem0)
        for r in range(STRIPE // CHUNK):
            pltpu.make_async_copy(
                rows1_v, acc_sh.at[pl.ds(sid * STRIPE + r * CHUNK, CHUNK)],
                semz).wait()
        plsc.subcore_barrier()

        for p in range(n_phases):
            if p > 0:
                pltpu.sync_copy(src_hbm.at[wid, pl.ds(p * PHASE, PHASE)],
                                src_v)
                pltpu.sync_copy(dst_hbm.at[wid, pl.ds(p * PHASE, PHASE)],
                                dst_v)
                pltpu.async_copy(y_hbm.at[src_v.at[0]], rows0_v, sem0)

            def body(i, carry):
                j = 2 * i
                pltpu.async_copy(y_hbm.at[src_v.at[j + 1]], rows1_v, sem1)
                pltpu.make_async_copy(
                    y_hbm.at[src_v.at[j]], rows0_v, sem0).wait()
                pltpu.sync_copy(rows0_v, acc_sh.at[dst_v.at[j]], add=True)
                pltpu.async_copy(y_hbm.at[src_v.at[j + 2]], rows0_v, sem0)
                pltpu.make_async_copy(
                    y_hbm.at[src_v.at[j + 1]], rows1_v, sem1).wait()
                pltpu.sync_copy(rows1_v, acc_sh.at[dst_v.at[j + 1]], add=True)
                return carry

            lax.fori_loop(0, (PHASE - 2) // 2, body, 0)
            pltpu.async_copy(y_hbm.at[src_v.at[PHASE - 1]], rows1_v, sem1)
            pltpu.make_async_copy(
                y_hbm.at[src_v.at[PHASE - 2]], rows0_v, sem0).wait()
            pltpu.sync_copy(rows0_v, acc_sh.at[dst_v.at[PHASE - 2]], add=True)
            pltpu.make_async_copy(
                y_hbm.at[src_v.at[PHASE - 1]], rows1_v, sem1).wait()
            pltpu.sync_copy(rows1_v, acc_sh.at[dst_v.at[PHASE - 1]], add=True)
        plsc.subcore_barrier()
        pltpu.sync_copy(acc_sh.at[pl.ds(sid * STRIPE, STRIPE)],
                        out_hbm.at[cid, pl.ds(sid * STRIPE, STRIPE)])

    return scatter_kernel(y, src_p, dst_p)


def _tc_first(x, W1, deg0, deg1):
    def body(x_ref, w_ref, d0_ref, d1_ref, y_ref, dis_ref):
        dis = lax.rsqrt(d0_ref[...] + d1_ref[...] + 1.0)
        dis_ref[...] = dis
        xw = jnp.dot(x_ref[...], w_ref[...], preferred_element_type=jnp.float32)
        y_ref[...] = xw * dis[:, None]

    return pl.pallas_call(
        body,
        out_shape=(jax.ShapeDtypeStruct((N, D), jnp.float32),
                   jax.ShapeDtypeStruct((N,), jnp.float32)),
    )(x, W1, deg0, deg1)


def _tc_mid(acc0, acc1, yprev, dis, b, W):
    def body(a0_ref, a1_ref, yp_ref, dis_ref, b_ref, w_ref, o_ref):
        dis = dis_ref[...][:, None]
        h = dis * (a0_ref[...] + a1_ref[...] + yp_ref[...]) + b_ref[...][None, :]
        h = jnp.maximum(h, 0.0)
        hw = jnp.dot(h, w_ref[...], preferred_element_type=jnp.float32)
        o_ref[...] = hw * dis

    return pl.pallas_call(
        body,
        out_shape=jax.ShapeDtypeStruct((N, D), jnp.float32),
    )(acc0, acc1, yprev, dis, b, W)


def _log_softmax(z):
    m = jnp.max(z, axis=1, keepdims=True)
    return z - (jnp.log(jnp.sum(jnp.exp(z - m), axis=1, keepdims=True)) + m)


def _tc_final(acc0, acc1, y3, dis, b3, prot, LW1, Lb1, LW2, Lb2):
    def body(a0_ref, a1_ref, y3_ref, dis_ref, b3_ref, prot_ref,
             lw1_ref, lb1_ref, lw2_ref, lb2_ref,
             out_ref, xrel_ref, oproto_ref):
        dis = dis_ref[...][:, None]
        h3 = dis * (a0_ref[...] + a1_ref[...] + y3_ref[...]) + b3_ref[...][None, :]
        qn = jnp.sqrt(jnp.sum(h3 * h3, axis=1, keepdims=True))
        hn = h3 / qn
        ids = lax.broadcasted_iota(jnp.int32, (NUM_CLASSES, N), 1)
        onehot = (ids == prot_ref[...][:, None]).astype(jnp.float32)
        anchors = jnp.dot(onehot, hn, preferred_element_type=jnp.float32)
        an = jnp.maximum(
            jnp.sqrt(jnp.sum(anchors * anchors, axis=1, keepdims=True)), 1e-6)
        xn = jnp.maximum(
            jnp.sqrt(jnp.sum(hn * hn, axis=1, keepdims=True)), 1e-6)
        xr = lax.dot_general(hn, anchors, (((1,), (1,)), ((), ())),
                             preferred_element_type=jnp.float32)
        xr = xr / (xn * an[:, 0][None, :])
        xrel_ref[...] = xr
        z = jnp.dot(anchors, lw1_ref[...], preferred_element_type=jnp.float32)
        z = jnp.maximum(z + lb1_ref[...][None, :], 0.0)
        z = jnp.dot(z, lw2_ref[...], preferred_element_type=jnp.float32)
        z = z + lb2_ref[...][None, :]
        op = _log_softmax(z)
        oproto_ref[...] = op
        o = jnp.dot(xr, op, preferred_element_type=jnp.float32)
        out_ref[...] = _log_softmax(o)

    return pl.pallas_call(
        body,
        out_shape=(jax.ShapeDtypeStruct((N, NUM_CLASSES), jnp.float32),
                   jax.ShapeDtypeStruct((N, NUM_CLASSES), jnp.float32),
                   jax.ShapeDtypeStruct((NUM_CLASSES, NUM_CLASSES), jnp.float32)),
    )(acc0, acc1, y3, dis, b3, prot, LW1, Lb1, LW2, Lb2)


def kernel(x, edge_index, epoch, prot, W1, b1, W2, b2, W3, b3,
           LW1, Lb1, LW2, Lb2):
    E = edge_index.shape[1]
    n_chunks = -(-E // (NW * CHUNK))
    n_chunks = -(-n_chunks // PHASE) * PHASE
    epad = NW * n_chunks * CHUNK
    src = edge_index[0]
    dst = edge_index[1]
    pad = epad - E
    pad_i = jnp.arange(pad, dtype=src.dtype)
    src_p = jnp.concatenate(
        [src, pad_i % N]).reshape(NW, n_chunks, CHUNK)
    dst_p = jnp.concatenate(
        [dst, N + pad_i % (NPAD - N)]).reshape(NW, n_chunks, CHUNK)

    degs = _sc_degree(dst_p, n_chunks)
    deg0 = degs[0, :N]
    deg1 = degs[1, :N]

    y1, dis = _tc_first(x, W1, deg0, deg1)
    acc = _sc_scatter(y1, src_p, dst_p, n_chunks)
    y2 = _tc_mid(acc[0, :N], acc[1, :N], y1, dis, b1, W2)
    acc = _sc_scatter(y2, src_p, dst_p, n_chunks)
    y3 = _tc_mid(acc[0, :N], acc[1, :N], y2, dis, b2, W3)
    acc = _sc_scatter(y3, src_p, dst_p, n_chunks)
    return _tc_final(acc[0, :N], acc[1, :N], y3, dis, b3, prot,
                     LW1, Lb1, LW2, Lb2)


# speedup vs baseline: 1.0275x; 1.0066x over previous
import functools

import jax
import jax.numpy as jnp
from jax import lax
from jax.experimental import pallas as pl
from jax.experimental.pallas import tpu as pltpu
from jax.experimental.pallas import tpu_sc as plsc

N = 10000
D = 128
NUM_CLASSES = 64
NC = 2
NS = 16
NW = NC * NS
CHUNK = 128
PHASE = 40
NPAD = 10240
STRIPE = NPAD // NS

_MESH = plsc.VectorSubcoreMesh(core_axis_name="c", subcore_axis_name="s")


def _sc_degree(dst_p, n_chunks):
    @functools.partial(
        pl.kernel,
        out_type=jax.ShapeDtypeStruct((NC, NPAD), jnp.float32),
        mesh=_MESH,
        scratch_types=[
            pltpu.VMEM((n_chunks, CHUNK), jnp.int32),
            pltpu.VMEM((CHUNK,), jnp.float32),
            pltpu.VMEM((STRIPE,), jnp.float32),
            pltpu.VMEM_SHARED((NPAD,), jnp.float32),
            pltpu.SemaphoreType.DMA,
        ],
    )
    def deg_kernel(dst_hbm, out_hbm, dst_v, ones_v, zb_v, acc_sh, sem):
        cid = lax.axis_index("c")
        sid = lax.axis_index("s")
        wid = sid * NC + cid
        for k in range(CHUNK // 16):
            ones_v[pl.ds(k * 16, 16)] = jnp.ones((16,), jnp.float32)

        def zfill(i, carry):
            zb_v[pl.ds(i * 16, 16)] = jnp.zeros((16,), jnp.float32)
            return carry

        lax.fori_loop(0, STRIPE // 16, zfill, 0)
        pltpu.sync_copy(dst_hbm.at[wid], dst_v)
        pltpu.sync_copy(zb_v, acc_sh.at[pl.ds(sid * STRIPE, STRIPE)])
        plsc.subcore_barrier()

        def body(j, carry):
            pltpu.async_copy(ones_v, acc_sh.at[dst_v.at[j]], sem, add=True)
            return carry

        lax.fori_loop(0, n_chunks, body, 0)

        def drain(j, carry):
            pltpu.make_async_copy(
                ones_v, acc_sh.at[dst_v.at[0]], sem).wait()
            return carry

        lax.fori_loop(0, n_chunks, drain, 0)
        plsc.subcore_barrier()
        pltpu.sync_copy(acc_sh.at[pl.ds(sid * STRIPE, STRIPE)],
                        out_hbm.at[cid, pl.ds(sid * STRIPE, STRIPE)])

    return deg_kernel(dst_p)


def _sc_scatter(y, src_p, dst_p, n_chunks):
    assert n_chunks % PHASE == 0
    n_phases = n_chunks // PHASE

    @functools.partial(
        pl.kernel,
        out_type=jax.ShapeDtypeStruct((NC, NPAD, D), jnp.float32),
        mesh=_MESH,
        scratch_types=[
            pltpu.VMEM((PHASE, CHUNK), jnp.int32),
            pltpu.VMEM((PHASE, CHUNK), jnp.int32),
            pltpu.VMEM((CHUNK, D), jnp.float32),
            pltpu.VMEM((CHUNK, D), jnp.float32),
            pltpu.VMEM_SHARED((NPAD, D), jnp.float32),
            pltpu.SemaphoreType.DMA,
            pltpu.SemaphoreType.DMA,
            pltpu.SemaphoreType.DMA,
        ],
    )
    def scatter_kernel(y_hbm, src_hbm, dst_hbm, out_hbm,
                       src_v, dst_v, rows0_v, rows1_v, acc_sh,
                       sem0, sem1, semz):
        cid = lax.axis_index("c")
        sid = lax.axis_index("s")
        wid = sid * NC + cid

        def zfill(i, carry):
            for k in range(D // 16):
                rows1_v[i, pl.ds(k * 16, 16)] = jnp.zeros((16,), jnp.float32)
            return carry

        lax.fori_loop(0, CHUNK, zfill, 0)
        for r in range(STRIPE // CHUNK):
            pltpu.async_copy(
                rows1_v, acc_sh.at[pl.ds(sid * STRIPE + r * CHUNK, CHUNK)],
                semz)
        pltpu.sync_copy(src_hbm.at[wid, pl.ds(0, PHASE)], src_v)
        pltpu.sync_copy(dst_hbm.at[wid, pl.ds(0, PHASE)], dst_v)
        pltpu.async_copy(y_hbm.at[src_v.at[0]], rows0_v, sem0)
        for r in range(STRIPE // CHUNK):
            pltpu.make_async_copy(
                rows1_v, acc_sh.at[pl.ds(sid * STRIPE + r * CHUNK, CHUNK)],
                semz).wait()
        plsc.subcore_barrier()

        for p in range(n_phases):
            if p > 0:
                pltpu.sync_copy(src_hbm.at[wid, pl.ds(p * PHASE, PHASE)],
                                src_v)
                pltpu.sync_copy(dst_hbm.at[wid, pl.ds(p * PHASE, PHASE)],
                                dst_v)
                pltpu.async_copy(y_hbm.at[src_v.at[0]], rows0_v, sem0)

            def body(i, carry):
                j = 2 * i
                pltpu.async_copy(y_hbm.at[src_v.at[j + 1]], rows1_v, sem1)
                pltpu.make_async_copy(
                    y_hbm.at[src_v.at[j]], rows0_v, sem0).wait()
                pltpu.sync_copy(rows0_v, acc_sh.at[dst_v.at[j]], add=True)
                pltpu.async_copy(y_hbm.at[src_v.at[j + 2]], rows0_v, sem0)
                pltpu.make_async_copy(
                    y_hbm.at[src_v.at[j + 1]], rows1_v, sem1).wait()
                pltpu.sync_copy(rows1_v, acc_sh.at[dst_v.at[j + 1]], add=True)
                return carry

            lax.fori_loop(0, (PHASE - 2) // 2, body, 0)
            pltpu.async_copy(y_hbm.at[src_v.at[PHASE - 1]], rows1_v, sem1)
            pltpu.make_async_copy(
                y_hbm.at[src_v.at[PHASE - 2]], rows0_v, sem0).wait()
            pltpu.sync_copy(rows0_v, acc_sh.at[dst_v.at[PHASE - 2]], add=True)
            pltpu.make_async_copy(
                y_hbm.at[src_v.at[PHASE - 1]], rows1_v, sem1).wait()
            pltpu.sync_copy(rows1_v, acc_sh.at[dst_v.at[PHASE - 1]], add=True)
        plsc.subcore_barrier()
        pltpu.sync_copy(acc_sh.at[pl.ds(sid * STRIPE, STRIPE)],
                        out_hbm.at[cid, pl.ds(sid * STRIPE, STRIPE)])

    return scatter_kernel(y, src_p, dst_p)


def _tc_first(x, W1, deg0, deg1):
    def body(x_ref, w_ref, d0_ref, d1_ref, y_ref, dis_ref):
        dis = lax.rsqrt(d0_ref[...] + d1_ref[...] + 1.0)
        dis_ref[...] = dis
        xw = jnp.dot(x_ref[...], w_ref[...], preferred_element_type=jnp.float32)
        y_ref[...] = xw * dis[:, None]

    return pl.pallas_call(
        body,
        out_shape=(jax.ShapeDtypeStruct((N, D), jnp.float32),
                   jax.ShapeDtypeStruct((N,), jnp.float32)),
    )(x, W1, deg0, deg1)


def _tc_mid(acc0, acc1, yprev, dis, b, W):
    def body(a0_ref, a1_ref, yp_ref, dis_ref, b_ref, w_ref, o_ref):
        dis = dis_ref[...][:, None]
        h = dis * (a0_ref[...] + a1_ref[...] + yp_ref[...]) + b_ref[...][None, :]
        h = jnp.maximum(h, 0.0)
        hw = jnp.dot(h, w_ref[...], preferred_element_type=jnp.float32)
        o_ref[...] = hw * dis

    return pl.pallas_call(
        body,
        out_shape=jax.ShapeDtypeStruct((N, D), jnp.float32),
    )(acc0, acc1, yprev, dis, b, W)


def _log_softmax(z):
    m = jnp.max(z, axis=1, keepdims=True)
    return z - (jnp.log(jnp.sum(jnp.exp(z - m), axis=1, keepdims=True)) + m)


def _tc_final(acc0, acc1, y3, dis, b3, prot, LW1, Lb1, LW2, Lb2):
    def body(a0_ref, a1_ref, y3_ref, dis_ref, b3_ref, prot_ref,
             lw1_ref, lb1_ref, lw2_ref, lb2_ref,
             out_ref, xrel_ref, oproto_ref):
        dis = dis_ref[...][:, None]
        h3 = dis * (a0_ref[...] + a1_ref[...] + y3_ref[...]) + b3_ref[...][None, :]
        qn = jnp.sqrt(jnp.sum(h3 * h3, axis=1, keepdims=True))
        hn = h3 / qn
        ids = lax.broadcasted_iota(jnp.int32, (NUM_CLASSES, N), 1)
        onehot = (ids == prot_ref[...][:, None]).astype(jnp.float32)
        anchors = jnp.dot(onehot, hn, preferred_element_type=jnp.float32)
        an = jnp.maximum(
            jnp.sqrt(jnp.sum(anchors * anchors, axis=1, keepdims=True)), 1e-6)
        xn = jnp.maximum(
            jnp.sqrt(jnp.sum(hn * hn, axis=1, keepdims=True)), 1e-6)
        xr = lax.dot_general(hn, anchors, (((1,), (1,)), ((), ())),
                             preferred_element_type=jnp.float32)
        xr = xr / (xn * an[:, 0][None, :])
        xrel_ref[...] = xr
        z = jnp.dot(anchors, lw1_ref[...], preferred_element_type=jnp.float32)
        z = jnp.maximum(z + lb1_ref[...][None, :], 0.0)
        z = jnp.dot(z, lw2_ref[...], preferred_element_type=jnp.float32)
        z = z + lb2_ref[...][None, :]
        op = _log_softmax(z)
        oproto_ref[...] = op
        o = jnp.dot(xr, op, preferred_element_type=jnp.float32)
        out_ref[...] = _log_softmax(o)

    return pl.pallas_call(
        body,
        out_shape=(jax.ShapeDtypeStruct((N, NUM_CLASSES), jnp.float32),
                   jax.ShapeDtypeStruct((N, NUM_CLASSES), jnp.float32),
                   jax.ShapeDtypeStruct((NUM_CLASSES, NUM_CLASSES), jnp.float32)),
    )(acc0, acc1, y3, dis, b3, prot, LW1, Lb1, LW2, Lb2)


def kernel(x, edge_index, epoch, prot, W1, b1, W2, b2, W3, b3,
           LW1, Lb1, LW2, Lb2):
    E = edge_index.shape[1]
    n_chunks = -(-E // (NW * CHUNK))
    n_chunks = -(-n_chunks // PHASE) * PHASE
    epad = NW * n_chunks * CHUNK
    src = edge_index[0]
    dst = edge_index[1]
    pad = epad - E
    pad_i = jnp.arange(pad, dtype=src.dtype)
    src_p = jnp.concatenate(
        [src, pad_i % N]).reshape(NW, n_chunks, CHUNK)
    dst_p = jnp.concatenate(
        [dst, N + pad_i % (NPAD - N)]).reshape(NW, n_chunks, CHUNK)

    degs = _sc_degree(dst_p, n_chunks)
    deg0 = degs[0, :N]
    deg1 = degs[1, :N]

    y1, dis = _tc_first(x, W1, deg0, deg1)
    acc = _sc_scatter(y1, src_p, dst_p, n_chunks)
    y2 = _tc_mid(acc[0, :N], acc[1, :N], y1, dis, b1, W2)
    acc = _sc_scatter(y2, src_p, dst_p, n_chunks)
    y3 = _tc_mid(acc[0, :N], acc[1, :N], y2, dis, b2, W3)
    acc = _sc_scatter(y3, src_p, dst_p, n_chunks)
    return _tc_final(acc[0, :N], acc[1, :N], y3, dis, b3, prot,
                     LW1, Lb1, LW2, Lb2)
